# baseline (speedup 1.0000x reference)
import jax
import jax.numpy as jnp
from jax import lax
from jax.experimental import pallas as pl
from jax.experimental.pallas import tpu as pltpu
from jax.experimental.pallas import tpu_sc as plsc

C = 64
NY = 496
NX = 432
B = 8
NP = 96000
NT = 32
TROWS = B * (NY // 8)
PCHUNK = 2000
NCH = NP // PCHUNK
CAP = 3840
NSCAP = 320
CB = 8
L = 16


def _iota():
    return lax.iota(jnp.int32, L)


def _sc_body(featsp_hbm, coords_hbm, out_hbm,
             cbuf, wcode, wp, sry, sx, sp, sdead, val, buf0, buf1, tmp,
             gsem, osem0, osem1):
    ci = lax.axis_index("c")
    si = lax.axis_index("s")
    wid = ci * 16 + si
    t0 = (wid * TROWS) >> 5
    t1 = ((wid + 1) * TROWS) >> 5
    iota = _iota()
    zeros16 = jnp.zeros((L,), jnp.float32)

    def _chunk(ch, n0):
        pltpu.sync_copy(coords_hbm.at[pl.ds(ch * PCHUNK * 4, PCHUNK * 4)],
                        cbuf)

        def _grp(j, n0):
            jv = j * L + iota
            jv4 = jv * 4
            s = plsc.load_gather(cbuf, [jv4])
            y = plsc.load_gather(cbuf, [jv4 + 2])
            x = plsc.load_gather(cbuf, [jv4 + 3])
            t = s * (NY // 8) + (y >> 3)
            m = (t >= t0) & (t < t1)
            code = (t << 12) | ((y & 7) << 9) | x
            pos = n0 + plsc.cumsum(m.astype(jnp.int32)) - 1
            plsc.store_scatter(wcode, [pos], code, mask=m)
            plsc.store_scatter(wp, [pos], ch * PCHUNK + jv, mask=m)
            return n0 + jnp.sum(m.astype(jnp.int32))
        return lax.fori_loop(0, PCHUNK // L, _grp, n0)
    n0 = lax.fori_loop(0, NCH, _chunk, jnp.int32(0))
    nv0 = (n0 + L - 1) >> 4

    bufs = (buf0, buf1)
    osems = (osem0, osem1)

    def _band(t, _):
        b = (t * 1058) >> 16
        ty = t - b * (NY // 8)

        def _sub(k, ns):
            lv = (k * L + iota) < n0
            wc = wcode[pl.ds(k * L, L)]
            p = wp[pl.ds(k * L, L)]
            m = lv & ((wc >> 12) == t)
            pos = ns + plsc.cumsum(m.astype(jnp.int32)) - 1
            plsc.store_scatter(sry, [pos], (wc >> 9) & 7, mask=m)
            plsc.store_scatter(sx, [pos], wc & 511, mask=m)
            plsc.store_scatter(sp, [pos], p, mask=m)
            return ns + jnp.sum(m.astype(jnp.int32))
        ns = lax.fori_loop(0, nv0, _sub, jnp.int32(0))
        ns16 = (ns + L - 1) >> 4

        def _kill(k, _):
            lv = (k * L + iota) < ns
            cell = (sry[pl.ds(k * L, L)] << 9) | sx[pl.ds(k * L, L)]
            cell = jnp.where(lv, cell, -1 - iota)
            tmp[...] = cell
            dead = jnp.zeros((L,), jnp.bool_)
            for r in range(1, L):
                rolled = plsc.load_gather(tmp, [(iota + r) & (L - 1)])
                dead = dead | ((rolled == cell) & (iota < L - r))
            sdead[pl.ds(k * L, L)] = (dead | jnp.logical_not(lv)).astype(
                jnp.int32)
            return 0
        lax.fori_loop(0, ns16, _kill, 0)

        lastv = jnp.full((L,), jnp.clip(ns - 1, 0, CAP - 1), jnp.int32)
        p_last = plsc.load_gather(sp, [lastv])
        ns32 = ((ns + 31) >> 5) << 5
        for k in range(2):
            pos = ns + k * L + iota
            m = pos < ns32
            plsc.store_scatter(sp, [jnp.clip(pos, 0, NSCAP - 1)], p_last,
                               mask=m)

        gds = []
        for i in range(NSCAP // 32):
            @pl.when(i * 32 < ns)
            def _fire(i=i):
                gds.append(pltpu.async_copy(
                    featsp_hbm.at[sp.at[pl.ds(i * 32, 32)]],
                    val.at[pl.ds(i * 32, 32)], gsem))
        for i in range(len(gds)):
            @pl.when(i * 32 < ns)
            def _drain(i=i):
                gds[i].wait()

        for cb in range(C // CB):
            par = cb & 1
            buf = bufs[par]
            osem = osems[par]

            def _wait_reuse(buf=buf, osem=osem):
                pltpu.make_async_copy(
                    out_hbm.at[0, pl.ds(0, CB), pl.ds(0, 8), :], buf,
                    osem).wait()
            if cb >= 2:
                _wait_reuse()
            else:
                @pl.when(t > t0)
                def _w():
                    _wait_reuse()

            def _ms(k, _, buf=buf):
                for j in range(CB):
                    for r in range(8):
                        buf[j, r, pl.ds(k * L, L)] = zeros16
                return 0
            lax.fori_loop(0, NX // L, _ms, 0)

            def _scat(v, _, buf=buf, cb=cb):
                ev = v * L + iota
                ry = sry[pl.ds(v * L, L)]
                xx = sx[pl.ds(v * L, L)]
                msk = sdead[pl.ds(v * L, L)] == 0
                for j in range(CB):
                    vals = plsc.load_gather(
                        val, [ev, jnp.full((L,), cb * CB + j, jnp.int32)])
                    plsc.store_scatter(
                        buf, [jnp.full((L,), j, jnp.int32), ry, xx], vals,
                        mask=msk)
                return 0
            lax.fori_loop(0, ns16, _scat, 0)

            pltpu.async_copy(
                buf, out_hbm.at[b, pl.ds(cb * CB, CB), pl.ds(ty * 8, 8), :],
                osem)
        return 0
    lax.fori_loop(t0, t1, _band, 0)

    for par in range(2):
        pltpu.make_async_copy(
            out_hbm.at[0, pl.ds(0, CB), pl.ds(0, 8), :], bufs[par],
            osems[par]).wait()


def _pad_body(f_ref, o_ref):
    o_ref[:, 0:C] = f_ref[...]
    o_ref[:, C:2 * C] = jnp.zeros_like(f_ref[...])


@jax.jit
def _pp_scatter(feats, coords):
    featsp = pl.pallas_call(
        _pad_body,
        grid=(NP // 4000,),
        in_specs=[pl.BlockSpec((4000, C), lambda i: (i, 0))],
        out_specs=pl.BlockSpec((4000, 2 * C), lambda i: (i, 0)),
        out_shape=jax.ShapeDtypeStruct((NP, 2 * C), jnp.float32),
    )(feats)
    mesh = plsc.VectorSubcoreMesh(core_axis_name="c", subcore_axis_name="s")
    run = pl.kernel(
        _sc_body,
        out_type=jax.ShapeDtypeStruct((B, C, NY, NX), jnp.float32),
        mesh=mesh,
        compiler_params=pltpu.CompilerParams(
            needs_layout_passes=False, use_tc_tiling_on_sc=True),
        scratch_types=[
            pltpu.VMEM((PCHUNK * 4,), jnp.int32),
            pltpu.VMEM((CAP,), jnp.int32),
            pltpu.VMEM((CAP,), jnp.int32),
            pltpu.VMEM((NSCAP,), jnp.int32),
            pltpu.VMEM((NSCAP,), jnp.int32),
            pltpu.VMEM((NSCAP,), jnp.int32),
            pltpu.VMEM((NSCAP,), jnp.int32),
            pltpu.VMEM((NSCAP, 2 * C), jnp.float32),
            pltpu.VMEM((CB, 8, NX), jnp.float32),
            pltpu.VMEM((CB, 8, NX), jnp.float32),
            pltpu.VMEM((L,), jnp.int32),
            pltpu.SemaphoreType.DMA,
            pltpu.SemaphoreType.DMA,
            pltpu.SemaphoreType.DMA,
        ],
    )
    return run(featsp, coords)


def kernel(batch_pillar_features_stacked, batch_coords, batch_size):
    feats = batch_pillar_features_stacked
    coords = batch_coords.astype(jnp.int32).reshape(-1)
    return _pp_scatter(feats, coords)


# speedup vs baseline: 1.0882x; 1.0882x over previous
import jax
import jax.numpy as jnp
from jax import lax
from jax.experimental import pallas as pl
from jax.experimental.pallas import tpu as pltpu
from jax.experimental.pallas import tpu_sc as plsc

C = 64
NY = 496
NX = 432
B = 8
NP = 96000
NT = 32
TROWS = B * (NY // 8)
PCHUNK = 2000
NCH = NP // PCHUNK
CAP = 3840
NSCAP = 320
CB = 8
L = 16


def _iota():
    return lax.iota(jnp.int32, L)


def _sc_body(featsp_hbm, coords_hbm, out_hbm,
             cbuf, wcode, wp, sry, sx, sp, sdead, val, buf0, buf1, tmp,
             gsem, osem0, osem1):
    ci = lax.axis_index("c")
    si = lax.axis_index("s")
    wid = ci * 16 + si
    t0 = (wid * TROWS) >> 5
    t1 = ((wid + 1) * TROWS) >> 5
    iota = _iota()
    zeros16 = jnp.zeros((L,), jnp.float32)

    def _chunk(ch, n0):
        pltpu.sync_copy(coords_hbm.at[pl.ds(ch * PCHUNK * 4, PCHUNK * 4)],
                        cbuf)

        def _grp(j, n0):
            jv = j * L + iota
            jv4 = jv * 4
            s = plsc.load_gather(cbuf, [jv4])
            y = plsc.load_gather(cbuf, [jv4 + 2])
            x = plsc.load_gather(cbuf, [jv4 + 3])
            t = s * (NY // 8) + (y >> 3)
            m = (t >= t0) & (t < t1)
            code = (t << 12) | ((y & 7) << 9) | x
            pos = n0 + plsc.cumsum(m.astype(jnp.int32)) - 1
            plsc.store_scatter(wcode, [pos], code, mask=m)
            plsc.store_scatter(wp, [pos], ch * PCHUNK + jv, mask=m)
            return n0 + jnp.sum(m.astype(jnp.int32))
        return lax.fori_loop(0, PCHUNK // L, _grp, n0)
    n0 = lax.fori_loop(0, NCH, _chunk, jnp.int32(0))
    nv0 = (n0 + L - 1) >> 4

    bufs = (buf0, buf1)
    osems = (osem0, osem1)

    def _ms0(k, _):
        for bb in range(2):
            for j in range(CB):
                for r in range(8):
                    bufs[bb][j, r, pl.ds(k * L, L)] = zeros16
        return 0
    lax.fori_loop(0, NX // L, _ms0, 0)

    def _band(t, ns_prev):
        b = (t * 1058) >> 16
        ty = t - b * (NY // 8)
        tb = t & 1
        tbv = jnp.full((L,), tb, jnp.int32)
        tbp = 1 - tb
        tbpv = jnp.full((L,), tbp, jnp.int32)

        def _sub(k, ns):
            lv = (k * L + iota) < n0
            wc = wcode[pl.ds(k * L, L)]
            p = wp[pl.ds(k * L, L)]
            m = lv & ((wc >> 12) == t)
            pos = ns + plsc.cumsum(m.astype(jnp.int32)) - 1
            plsc.store_scatter(sry, [tbv, pos], (wc >> 9) & 7, mask=m)
            plsc.store_scatter(sx, [tbv, pos], wc & 511, mask=m)
            plsc.store_scatter(sp, [pos], p, mask=m)
            return ns + jnp.sum(m.astype(jnp.int32))
        ns = lax.fori_loop(0, nv0, _sub, jnp.int32(0))
        ns16 = (ns + L - 1) >> 4

        def _kill(k, _):
            lv = (k * L + iota) < ns
            cell = (sry[tb, pl.ds(k * L, L)] << 9) | sx[tb, pl.ds(k * L, L)]
            cell = jnp.where(lv, cell, -1 - iota)
            tmp[...] = cell
            dead = jnp.zeros((L,), jnp.bool_)
            for r in range(1, L):
                rolled = plsc.load_gather(tmp, [(iota + r) & (L - 1)])
                dead = dead | ((rolled == cell) & (iota < L - r))
            sdead[tb, pl.ds(k * L, L)] = (dead | jnp.logical_not(lv)).astype(
                jnp.int32)
            return 0
        lax.fori_loop(0, ns16, _kill, 0)

        lastv = jnp.full((L,), jnp.clip(ns - 1, 0, CAP - 1), jnp.int32)
        p_last = plsc.load_gather(sp, [lastv])
        ns32 = ((ns + 31) >> 5) << 5
        for k in range(2):
            pos = ns + k * L + iota
            m = pos < ns32
            plsc.store_scatter(sp, [jnp.clip(pos, 0, NSCAP - 1)], p_last,
                               mask=m)

        gds = []
        for i in range(NSCAP // 32):
            @pl.when(i * 32 < ns)
            def _fire(i=i):
                gds.append(pltpu.async_copy(
                    featsp_hbm.at[sp.at[pl.ds(i * 32, 32)]],
                    val.at[pl.ds(i * 32, 32)], gsem))
        for i in range(len(gds)):
            @pl.when(i * 32 < ns)
            def _drain(i=i):
                gds[i].wait()

        for cb in range(C // CB):
            par = cb & 1
            buf = bufs[par]
            osem = osems[par]

            def _wait_reuse(buf=buf, osem=osem):
                pltpu.make_async_copy(
                    out_hbm.at[0, pl.ds(0, CB), pl.ds(0, 8), :], buf,
                    osem).wait()
            if cb >= 2:
                _wait_reuse()
            else:
                @pl.when(t > t0)
                def _w():
                    _wait_reuse()

                def _undo(v, _, buf=buf):
                    ry = sry[tbp, pl.ds(v * L, L)]
                    xx = sx[tbp, pl.ds(v * L, L)]
                    msk = sdead[tbp, pl.ds(v * L, L)] == 0
                    for j in range(CB):
                        plsc.store_scatter(
                            buf, [jnp.full((L,), j, jnp.int32), ry, xx],
                            zeros16, mask=msk)
                    return 0
                lax.fori_loop(0, (ns_prev + L - 1) >> 4, _undo, 0)

            def _scat(v, _, buf=buf, cb=cb):
                ev = v * L + iota
                ry = sry[tb, pl.ds(v * L, L)]
                xx = sx[tb, pl.ds(v * L, L)]
                msk = sdead[tb, pl.ds(v * L, L)] == 0
                for j in range(CB):
                    vals = plsc.load_gather(
                        val, [ev, jnp.full((L,), cb * CB + j, jnp.int32)])
                    plsc.store_scatter(
                        buf, [jnp.full((L,), j, jnp.int32), ry, xx], vals,
                        mask=msk)
                return 0
            lax.fori_loop(0, ns16, _scat, 0)

            pltpu.async_copy(
                buf, out_hbm.at[b, pl.ds(cb * CB, CB), pl.ds(ty * 8, 8), :],
                osem)
        return ns
    lax.fori_loop(t0, t1, _band, jnp.int32(0))

    for par in range(2):
        pltpu.make_async_copy(
            out_hbm.at[0, pl.ds(0, CB), pl.ds(0, 8), :], bufs[par],
            osems[par]).wait()


def _pad_body(f_ref, o_ref):
    o_ref[:, 0:C] = f_ref[...]
    o_ref[:, C:2 * C] = jnp.zeros_like(f_ref[...])


@jax.jit
def _pp_scatter(feats, coords):
    featsp = pl.pallas_call(
        _pad_body,
        grid=(NP // 4000,),
        in_specs=[pl.BlockSpec((4000, C), lambda i: (i, 0))],
        out_specs=pl.BlockSpec((4000, 2 * C), lambda i: (i, 0)),
        out_shape=jax.ShapeDtypeStruct((NP, 2 * C), jnp.float32),
    )(feats)
    mesh = plsc.VectorSubcoreMesh(core_axis_name="c", subcore_axis_name="s")
    run = pl.kernel(
        _sc_body,
        out_type=jax.ShapeDtypeStruct((B, C, NY, NX), jnp.float32),
        mesh=mesh,
        compiler_params=pltpu.CompilerParams(
            needs_layout_passes=False, use_tc_tiling_on_sc=True),
        scratch_types=[
            pltpu.VMEM((PCHUNK * 4,), jnp.int32),
            pltpu.VMEM((CAP,), jnp.int32),
            pltpu.VMEM((CAP,), jnp.int32),
            pltpu.VMEM((2, NSCAP), jnp.int32),
            pltpu.VMEM((2, NSCAP), jnp.int32),
            pltpu.VMEM((NSCAP,), jnp.int32),
            pltpu.VMEM((2, NSCAP), jnp.int32),
            pltpu.VMEM((NSCAP, 2 * C), jnp.float32),
            pltpu.VMEM((CB, 8, NX), jnp.float32),
            pltpu.VMEM((CB, 8, NX), jnp.float32),
            pltpu.VMEM((L,), jnp.int32),
            pltpu.SemaphoreType.DMA,
            pltpu.SemaphoreType.DMA,
            pltpu.SemaphoreType.DMA,
        ],
    )
    return run(featsp, coords)


def kernel(batch_pillar_features_stacked, batch_coords, batch_size):
    feats = batch_pillar_features_stacked
    coords = batch_coords.astype(jnp.int32).reshape(-1)
    return _pp_scatter(feats, coords)


# speedup vs baseline: 1.0887x; 1.0004x over previous
import jax
import jax.numpy as jnp
from jax import lax
from jax.experimental import pallas as pl
from jax.experimental.pallas import tpu as pltpu
from jax.experimental.pallas import tpu_sc as plsc

C = 64
NY = 496
NX = 432
B = 8
NP = 96000
NT = 32
TROWS = B * (NY // 8)
PCHUNK = 2000
NCH = NP // PCHUNK
CAP = 3840
NSCAP = 320
CB = 8
L = 16


def _iota():
    return lax.iota(jnp.int32, L)


def _sc_body(featsp_hbm, coords_hbm, out_hbm,
             cbuf, wcode, wp, sry, sx, sp, sdead, val, buf0, buf1, tmp,
             gsem, osem0, osem1):
    ci = lax.axis_index("c")
    si = lax.axis_index("s")
    wid = ci * 16 + si
    t0 = (wid * TROWS) >> 5
    t1 = ((wid + 1) * TROWS) >> 5
    iota = _iota()
    zeros16 = jnp.zeros((L,), jnp.float32)

    def _chunk(ch, n0):
        pltpu.sync_copy(coords_hbm.at[pl.ds(ch * PCHUNK * 4, PCHUNK * 4)],
                        cbuf)

        def _grp(j, n0):
            jv = j * L + iota
            jv4 = jv * 4
            s = plsc.load_gather(cbuf, [jv4])
            y = plsc.load_gather(cbuf, [jv4 + 2])
            x = plsc.load_gather(cbuf, [jv4 + 3])
            t = s * (NY // 8) + (y >> 3)
            m = (t >= t0) & (t < t1)
            code = (t << 12) | ((y & 7) << 9) | x
            pos = n0 + plsc.cumsum(m.astype(jnp.int32)) - 1
            plsc.store_scatter(wcode, [pos], code, mask=m)
            plsc.store_scatter(wp, [pos], ch * PCHUNK + jv, mask=m)
            return n0 + jnp.sum(m.astype(jnp.int32))
        return lax.fori_loop(0, PCHUNK // L, _grp, n0)
    n0 = lax.fori_loop(0, NCH, _chunk, jnp.int32(0))
    nv0 = (n0 + L - 1) >> 4

    bufs = (buf0, buf1)
    osems = (osem0, osem1)

    def _ms0(k, _):
        for bb in range(2):
            for j in range(CB):
                for r in range(8):
                    bufs[bb][j, r, pl.ds(k * L, L)] = zeros16
        return 0
    lax.fori_loop(0, NX // L, _ms0, 0)

    def _band(t, ns_prev):
        b = (t * 1058) >> 16
        ty = t - b * (NY // 8)
        tb = t & 1
        tbv = jnp.full((L,), tb, jnp.int32)
        tbp = 1 - tb

        def _sub(k, ns):
            lv = (k * L + iota) < n0
            wc = wcode[pl.ds(k * L, L)]
            p = wp[pl.ds(k * L, L)]
            m = lv & ((wc >> 12) == t)
            pos = ns + plsc.cumsum(m.astype(jnp.int32)) - 1
            plsc.store_scatter(sry, [tbv, pos], (wc >> 9) & 7, mask=m)
            plsc.store_scatter(sx, [tbv, pos], wc & 511, mask=m)
            plsc.store_scatter(sp, [pos], p, mask=m)
            return ns + jnp.sum(m.astype(jnp.int32))
        ns = lax.fori_loop(0, nv0, _sub, jnp.int32(0))
        ns16 = (ns + L - 1) >> 4

        def _kill(k, _):
            lv = (k * L + iota) < ns
            cell = (sry[tb, pl.ds(k * L, L)] << 9) | sx[tb, pl.ds(k * L, L)]
            cell = jnp.where(lv, cell, -1 - iota)
            tmp[...] = cell
            dead = jnp.zeros((L,), jnp.bool_)
            for r in range(1, L):
                rolled = plsc.load_gather(tmp, [(iota + r) & (L - 1)])
                dead = dead | ((rolled == cell) & (iota < L - r))
            sdead[tb, pl.ds(k * L, L)] = (dead | jnp.logical_not(lv)).astype(
                jnp.int32)
            return 0
        lax.fori_loop(0, ns16, _kill, 0)

        lastv = jnp.full((L,), jnp.clip(ns - 1, 0, CAP - 1), jnp.int32)
        p_last = plsc.load_gather(sp, [lastv])
        ns32 = ((ns + 31) >> 5) << 5
        for k in range(2):
            pos = ns + k * L + iota
            m = pos < ns32
            plsc.store_scatter(sp, [jnp.clip(pos, 0, NSCAP - 1)], p_last,
                               mask=m)

        gds = []
        for i in range(NSCAP // 32):
            @pl.when(i * 32 < ns)
            def _fire(i=i):
                gds.append(pltpu.async_copy(
                    featsp_hbm.at[sp.at[pl.ds(i * 32, 32)]],
                    val.at[pl.ds(i * 32, 32)], gsem))
        for i in range(len(gds)):
            @pl.when(i * 32 < ns)
            def _drain(i=i):
                gds[i].wait()

        for cb in range(C // CB):
            par = cb & 1
            buf = bufs[par]
            osem = osems[par]

            def _wait_reuse(buf=buf, osem=osem):
                pltpu.make_async_copy(
                    out_hbm.at[0, pl.ds(0, CB), pl.ds(0, 8), :], buf,
                    osem).wait()
            if cb >= 2:
                _wait_reuse()
            else:
                @pl.when(t > t0)
                def _w():
                    _wait_reuse()

                def _undo(v, _, buf=buf):
                    ry = sry[tbp, pl.ds(v * L, L)]
                    xx = sx[tbp, pl.ds(v * L, L)]
                    msk = sdead[tbp, pl.ds(v * L, L)] == 0
                    for j in range(CB):
                        plsc.store_scatter(
                            buf, [jnp.full((L,), j, jnp.int32), ry, xx],
                            zeros16, mask=msk)
                    return 0
                lax.fori_loop(0, (ns_prev + L - 1) >> 4, _undo, 0)

            def _scat(v, _, buf=buf, cb=cb):
                ev = v * L + iota
                ry = sry[tb, pl.ds(v * L, L)]
                xx = sx[tb, pl.ds(v * L, L)]
                msk = sdead[tb, pl.ds(v * L, L)] == 0
                for j in range(CB):
                    vals = plsc.load_gather(
                        val, [ev, jnp.full((L,), cb * CB + j, jnp.int32)])
                    plsc.store_scatter(
                        buf, [jnp.full((L,), j, jnp.int32), ry, xx], vals,
                        mask=msk)
                return 0
            lax.fori_loop(0, ns16, _scat, 0)

            pltpu.async_copy(
                buf, out_hbm.at[b, pl.ds(cb * CB, CB), pl.ds(ty * 8, 8), :],
                osem)
        return ns
    lax.fori_loop(t0, t1, _band, jnp.int32(0))

    for par in range(2):
        pltpu.make_async_copy(
            out_hbm.at[0, pl.ds(0, CB), pl.ds(0, 8), :], bufs[par],
            osems[par]).wait()


def _pad_body(f_ref, o_ref):
    o_ref[:, 0:C] = f_ref[...]
    o_ref[:, C:2 * C] = jnp.zeros_like(f_ref[...])


@jax.jit
def _pp_scatter(feats, coords):
    featsp = pl.pallas_call(
        _pad_body,
        grid=(NP // 4000,),
        in_specs=[pl.BlockSpec((4000, C), lambda i: (i, 0))],
        out_specs=pl.BlockSpec((4000, 2 * C), lambda i: (i, 0)),
        out_shape=jax.ShapeDtypeStruct((NP, 2 * C), jnp.float32),
    )(feats)
    mesh = plsc.VectorSubcoreMesh(core_axis_name="c", subcore_axis_name="s")
    run = pl.kernel(
        _sc_body,
        out_type=jax.ShapeDtypeStruct((B, C, NY, NX), jnp.float32),
        mesh=mesh,
        compiler_params=pltpu.CompilerParams(
            needs_layout_passes=False, use_tc_tiling_on_sc=True),
        scratch_types=[
            pltpu.VMEM((PCHUNK * 4,), jnp.int32),
            pltpu.VMEM((CAP,), jnp.int32),
            pltpu.VMEM((CAP,), jnp.int32),
            pltpu.VMEM((2, NSCAP), jnp.int32),
            pltpu.VMEM((2, NSCAP), jnp.int32),
            pltpu.VMEM((NSCAP,), jnp.int32),
            pltpu.VMEM((2, NSCAP), jnp.int32),
            pltpu.VMEM((NSCAP, 2 * C), jnp.float32),
            pltpu.VMEM((CB, 8, NX), jnp.float32),
            pltpu.VMEM((CB, 8, NX), jnp.float32),
            pltpu.VMEM((L,), jnp.int32),
            pltpu.SemaphoreType.DMA,
            pltpu.SemaphoreType.DMA,
            pltpu.SemaphoreType.DMA,
        ],
    )
    return run(featsp, coords)


def kernel(batch_pillar_features_stacked, batch_coords, batch_size):
    feats = batch_pillar_features_stacked
    coords = batch_coords.astype(jnp.int32).reshape(-1)
    return _pp_scatter(feats, coords)
